# baseline (speedup 1.0000x reference)
import jax
import jax.numpy as jnp
from jax import lax
from jax.experimental import pallas as pl
from jax.experimental.pallas import tpu as pltpu
from jax.experimental.pallas import tpu_sc as plsc

N_NODES = 10000
MAX_DEG = 32
D_FEAT = 128
_NW = 32
_C = 312
_T = 8
_NT = _C // _T
_TAIL0 = _NW * _C
_INV = 1.0 / MAX_DEG


def _reduce_tile(buf, obuf):
    def per_node(n, carry):
        for c in range(D_FEAT // 16):
            sl = pl.ds(c * 16, 16)
            acc = buf[n, 0, sl]
            for k in range(1, MAX_DEG):
                acc = acc + buf[n, k, sl]
            obuf[n, sl] = acc * _INV
        return carry
    lax.fori_loop(0, _T, per_node, 0)


def _sc_body(mail, out, buf0, buf1, ob, sem0, sem1):
    w = lax.axis_index("s") * 2 + lax.axis_index("c")
    base = w * _C
    bufs = (buf0, buf1)
    sems = (sem0, sem1)
    pltpu.async_copy(mail.at[pl.ds(base, _T)], buf0, sem0)
    pltpu.async_copy(mail.at[pl.ds(base + _T, _T)], buf1, sem1)

    def pair(i, carry):
        t0 = i * 2
        for b in range(2):
            t = t0 + b
            node0 = base + t * _T
            pltpu.make_async_copy(mail.at[pl.ds(node0, _T)], bufs[b], sems[b]).wait()
            _reduce_tile(bufs[b], ob)
            pltpu.sync_copy(ob, out.at[pl.ds(node0, _T)])

            @pl.when(t + 2 < _NT)
            def _():
                pltpu.async_copy(
                    mail.at[pl.ds(node0 + 2 * _T, _T)], bufs[b], sems[b])
        return carry

    lax.fori_loop(0, _NT // 2, pair, 0)

    last0 = base + (_NT - 1) * _T
    pltpu.make_async_copy(mail.at[pl.ds(last0, _T)], buf0, sem0).wait()
    _reduce_tile(buf0, ob)
    pltpu.sync_copy(ob, out.at[pl.ds(last0, _T)])

    @pl.when(w < 2)
    def _():
        node0 = _TAIL0 + w * _T
        pltpu.sync_copy(mail.at[pl.ds(node0, _T)], buf1)
        _reduce_tile(buf1, ob)
        pltpu.sync_copy(ob, out.at[pl.ds(node0, _T)])


def kernel(mailbox_m):
    mesh = plsc.VectorSubcoreMesh(core_axis_name="c", subcore_axis_name="s")
    f = pl.kernel(
        _sc_body,
        out_type=jax.ShapeDtypeStruct((N_NODES, D_FEAT), jnp.float32),
        mesh=mesh,
        scratch_types=[
            pltpu.VMEM((_T, MAX_DEG, D_FEAT), jnp.float32),
            pltpu.VMEM((_T, MAX_DEG, D_FEAT), jnp.float32),
            pltpu.VMEM((_T, D_FEAT), jnp.float32),
            pltpu.SemaphoreType.DMA,
            pltpu.SemaphoreType.DMA,
        ],
    )
    return f(mailbox_m)
